# baseline (speedup 1.0000x reference)
import numpy as np
import jax
import jax.numpy as jnp
from jax import lax
from jax.experimental import pallas as pl
from jax.experimental.pallas import tpu as pltpu
from jax.experimental.pallas import tpu_sc as plsc

_NUM_LEVELS = 12
_FEAT = 8
_TABLE = 2 ** 19
_MASK = _TABLE - 1
_BASE_RES = 16
_MAX_RES = 512
_N_PTS = 65536
_HIDDEN = 64

_growth = np.exp((np.log(_MAX_RES) - np.log(_BASE_RES)) / (_NUM_LEVELS - 1))
_RES = [int(np.floor(_BASE_RES * _growth ** l)) for l in range(_NUM_LEVELS)]
_K1 = int(np.uint32(2654435761).astype(np.int32))
_K2 = 805459861

_NC, _NS = 2, 16
_NW = _NC * _NS
_P = 128
_PTS_PER_W = _N_PTS // _NW
_NCHUNK = _PTS_PER_W // _P
_R = _NUM_LEVELS * 8
_ENC = _NUM_LEVELS * _FEAT
_FB = _P * _ENC


def _sc_encode_body(coords_hbm, ht_hbm, out_hbm,
                    crd_v, fr_v, idx_v, rows_v, feats_v, gsem, csem, wsem):
    wid = lax.axis_index("s") * _NC + lax.axis_index("c")
    wbase = wid * _PTS_PER_W
    lane = lax.iota(jnp.int32, 16)
    pat8 = lax.shift_right_logical(lane, 3)
    lane7 = lane & 7

    def fire_coords(c):
        for d in range(3):
            pltpu.async_copy(
                coords_hbm.at[pl.ds(d * _N_PTS + wbase + c * _P, _P)],
                crd_v.at[pl.ds((c & 1) * (3 * _P) + d * _P, _P)], csem)

    def drain_coords():
        for d in range(3):
            pltpu.make_async_copy(coords_hbm.at[pl.ds(0, _P)],
                                  crd_v.at[pl.ds(d * _P, _P)], csem).wait()

    fire_coords(0)

    def chunk_body(chunk, carry):
        base = wbase + chunk * _P
        cb = (chunk & 1) * (3 * _P)
        drain_coords()
        fire_coords(jnp.minimum(chunk + 1, _NCHUNK - 1))

        def grp(g, c2):
            off = g * 16
            xg = plsc.load_gather(crd_v, [cb + off + lane])
            yg = plsc.load_gather(crd_v, [cb + _P + off + lane])
            zg = plsc.load_gather(crd_v, [cb + 2 * _P + off + lane])
            for l in range(_NUM_LEVELS):
                res = float(_RES[l])
                fx = xg * res
                xi = fx.astype(jnp.int32)
                plsc.store_scatter(fr_v, [(l * 3 + 0) * _P + off + lane],
                                   fx - xi.astype(jnp.float32))
                fy = yg * res
                yi = fy.astype(jnp.int32)
                plsc.store_scatter(fr_v, [(l * 3 + 1) * _P + off + lane],
                                   fy - yi.astype(jnp.float32))
                fz = zg * res
                zi = fz.astype(jnp.int32)
                plsc.store_scatter(fr_v, [(l * 3 + 2) * _P + off + lane],
                                   fz - zi.astype(jnp.float32))
                x1 = xi + 1
                y0 = yi * _K1
                y1 = y0 + _K1
                z0 = zi * _K2
                z1 = z0 + _K2
                c = 0
                for hx in (xi, x1):
                    for hy in (y0, y1):
                        for hz in (z0, z1):
                            h = hx ^ hy ^ hz
                            plsc.store_scatter(
                                idx_v, [(l * 8 + c) * _P + off + lane],
                                h & _MASK)
                            c += 1
            return c2
        lax.fori_loop(0, _P // 16, grp, None)

        handles = [pltpu.async_copy(
                       ht_hbm.at[r // 8].at[idx_v.at[pl.ds(r * _P, _P)]],
                       rows_v.at[pl.ds(r * _P, _P)], gsem)
                   for r in range(_R)]
        for h in handles:
            h.wait()

        @pl.when(chunk > 0)
        def _():
            pltpu.make_async_copy(out_hbm.at[pl.ds(0, _FB)], feats_v,
                                  wsem).wait()

        def pair(k, c2):
            i1 = k * 2 + pat8
            fb = i1 * _ENC + lane7
            for l in range(_NUM_LEVELS):
                wx = plsc.load_gather(fr_v, [(l * 3 + 0) * _P + i1])
                wy = plsc.load_gather(fr_v, [(l * 3 + 1) * _P + i1])
                wz = plsc.load_gather(fr_v, [(l * 3 + 2) * _P + i1])
                wx0 = 1.0 - wx
                wy0 = 1.0 - wy
                wz0 = 1.0 - wz
                c00 = wy0 * wz0
                c01 = wy0 * wz
                c10 = wy * wz0
                c11 = wy * wz
                wts = (wx0 * c00, wx0 * c01, wx0 * c10, wx0 * c11,
                       wx * c00, wx * c01, wx * c10, wx * c11)
                acc = None
                for c in range(8):
                    g = plsc.load_gather(rows_v,
                                         [(l * 8 + c) * _P + i1, lane7])
                    t = g * wts[c]
                    acc = t if acc is None else acc + t
                plsc.store_scatter(feats_v, [fb + l * 8], acc)
            return c2
        lax.fori_loop(0, _P // 2, pair, None)

        pltpu.async_copy(feats_v, out_hbm.at[pl.ds(base * _ENC, _FB)], wsem)
        return carry

    lax.fori_loop(0, _NCHUNK, chunk_body, None)
    pltpu.make_async_copy(out_hbm.at[pl.ds(0, _FB)], feats_v, wsem).wait()
    drain_coords()


def _sc_encode(coords_flat, ht):
    mesh = plsc.VectorSubcoreMesh(core_axis_name="c", subcore_axis_name="s",
                                  num_cores=_NC, num_subcores=_NS)
    f = pl.kernel(
        _sc_encode_body,
        out_type=jax.ShapeDtypeStruct((_N_PTS * _ENC,), jnp.float32),
        mesh=mesh,
        scratch_types=[
            pltpu.VMEM((2 * 3 * _P,), jnp.float32),
            pltpu.VMEM((_NUM_LEVELS * 3 * _P,), jnp.float32),
            pltpu.VMEM((_R * _P,), jnp.int32),
            pltpu.VMEM((_R * _P, _FEAT), jnp.float32),
            pltpu.VMEM((_FB,), jnp.float32),
            pltpu.SemaphoreType.DMA,
            pltpu.SemaphoreType.DMA,
            pltpu.SemaphoreType.DMA,
        ],
        compiler_params=pltpu.CompilerParams(needs_layout_passes=False,
                                             use_tc_tiling_on_sc=False),
    )
    return f(coords_flat, ht)


def _mlp(feats, view_dirs, W1, b1, W2, b2, W3, b3):
    BT = 8192
    W1f = W1[:_ENC]
    W1v = W1[_ENC:]

    def body(f_ref, vd_ref, w1f, w1v, b1r, w2, b2r, w3, b3r, out_ref):
        hp = lax.Precision.HIGHEST
        f = f_ref[...]
        vd = vd_ref[...]
        h = jnp.dot(f, w1f[...], preferred_element_type=jnp.float32,
                    precision=hp)
        h = h + jnp.dot(vd, w1v[...], preferred_element_type=jnp.float32,
                        precision=hp)
        h = jnp.maximum(h + b1r[...], 0.0)
        h = jnp.dot(h, w2[...], preferred_element_type=jnp.float32,
                    precision=hp)
        h = jnp.maximum(h + b2r[...], 0.0)
        raw = jnp.dot(h, w3[...], preferred_element_type=jnp.float32,
                      precision=hp) + b3r[...]
        ci = lax.broadcasted_iota(jnp.int32, raw.shape, 1)
        out_ref[...] = jnp.where(ci < 3, jax.nn.sigmoid(raw),
                                 jnp.maximum(raw, 0.0))

    return pl.pallas_call(
        body,
        grid=(_N_PTS // BT,),
        in_specs=[
            pl.BlockSpec((BT, _ENC), lambda i: (i, 0)),
            pl.BlockSpec((BT, 3), lambda i: (i, 0)),
            pl.BlockSpec((_ENC, _HIDDEN), lambda i: (0, 0)),
            pl.BlockSpec((3, _HIDDEN), lambda i: (0, 0)),
            pl.BlockSpec((1, _HIDDEN), lambda i: (0, 0)),
            pl.BlockSpec((_HIDDEN, _HIDDEN), lambda i: (0, 0)),
            pl.BlockSpec((1, _HIDDEN), lambda i: (0, 0)),
            pl.BlockSpec((_HIDDEN, 4), lambda i: (0, 0)),
            pl.BlockSpec((1, 4), lambda i: (0, 0)),
        ],
        out_specs=pl.BlockSpec((BT, 4), lambda i: (i, 0)),
        out_shape=jax.ShapeDtypeStruct((_N_PTS, 4), jnp.float32),
    )(feats, view_dirs, W1f, W1v, b1.reshape(1, -1), W2, b2.reshape(1, -1),
      W3, b3.reshape(1, -1))


def kernel(coords, view_dirs, hash_tables, W1, b1, W2, b2, W3, b3):
    coords_flat = coords.T.reshape(-1)
    feats = _sc_encode(coords_flat, hash_tables).reshape(_N_PTS, _ENC)
    out = _mlp(feats, view_dirs, W1, b1, W2, b2, W3, b3)
    return out[:, :3], out[:, 3:4]


# speedup vs baseline: 2.0168x; 2.0168x over previous
import numpy as np
import jax
import jax.numpy as jnp
from jax import lax
from jax.experimental import pallas as pl
from jax.experimental.pallas import tpu as pltpu
from jax.experimental.pallas import tpu_sc as plsc

_NUM_LEVELS = 12
_FEAT = 8
_TABLE = 2 ** 19
_MASK = _TABLE - 1
_BASE_RES = 16
_MAX_RES = 512
_N_PTS = 65536
_HIDDEN = 64

_growth = np.exp((np.log(_MAX_RES) - np.log(_BASE_RES)) / (_NUM_LEVELS - 1))
_RES = [int(np.floor(_BASE_RES * _growth ** l)) for l in range(_NUM_LEVELS)]
_K1 = int(np.uint32(2654435761).astype(np.int32))
_K2 = 805459861

_NC, _NS = 2, 16
_NW = _NC * _NS
_P = 128
_PTS_PER_W = _N_PTS // _NW
_NCHUNK = _PTS_PER_W // _P
_R = _NUM_LEVELS * 8
_ENC = _NUM_LEVELS * _FEAT
_FB = _P * _ENC

_TILES = _NUM_LEVELS * (_TABLE // 128)
_TPW = _TILES // _NW
_KT = 24
_NIT = _TPW // _KT


def _sc_retile_body(src_hbm, out_hbm, bin_v, bout_v, csem, wsem):
    wid = lax.axis_index("s") * _NC + lax.axis_index("c")
    lane = lax.iota(jnp.int32, 16)
    pat8 = lax.shift_right_logical(lane, 3)
    lane7 = lane & 7
    patf = lane7 * 128 + pat8
    t0w = wid * _TPW

    def fire(it):
        pltpu.async_copy(
            src_hbm.at[pl.ds((t0w + it * _KT) * 1024, _KT * 1024)],
            bin_v.at[pl.ds((it & 1) * (_KT * 1024), _KT * 1024)], csem)

    fire(0)

    def it_body(it, carry):
        pltpu.make_async_copy(src_hbm.at[pl.ds(0, _KT * 1024)],
                              bin_v.at[pl.ds(0, _KT * 1024)], csem).wait()
        fire(jnp.minimum(it + 1, _NIT - 1))
        bb = (it & 1) * (_KT * 1024)
        ob = (it & 1) * (_KT * 128)

        @pl.when(it > 1)
        def _():
            pltpu.make_async_copy(
                out_hbm.at[pl.ds(0, _KT * 128)],
                bout_v.at[pl.ds(0, _KT * 128), :], wsem).wait()

        def tile_body(k, c2):
            kb = bb + k * 1024
            orow = ob + k * 128
            for c0 in range(0, 128, 2):
                v = plsc.load_gather(bin_v, [kb + c0 + patf])
                plsc.store_scatter(bout_v, [orow + c0 + pat8, lane7], v)
            return c2
        lax.fori_loop(0, _KT, tile_body, None)

        pltpu.async_copy(
            bout_v.at[pl.ds(ob, _KT * 128), :],
            out_hbm.at[pl.ds((t0w + it * _KT) * 128, _KT * 128)], wsem)
        return carry

    lax.fori_loop(0, _NIT, it_body, None)
    for _ in range(2):
        pltpu.make_async_copy(out_hbm.at[pl.ds(0, _KT * 128)],
                              bout_v.at[pl.ds(0, _KT * 128), :], wsem).wait()
    pltpu.make_async_copy(src_hbm.at[pl.ds(0, _KT * 1024)],
                          bin_v.at[pl.ds(0, _KT * 1024)], csem).wait()


def _sc_retile(src_flat):
    mesh = plsc.VectorSubcoreMesh(core_axis_name="c", subcore_axis_name="s",
                                  num_cores=_NC, num_subcores=_NS)
    f = pl.kernel(
        _sc_retile_body,
        out_type=jax.ShapeDtypeStruct((_NUM_LEVELS * _TABLE, _FEAT),
                                      jnp.float32),
        mesh=mesh,
        scratch_types=[
            pltpu.VMEM((2 * _KT * 1024,), jnp.float32),
            pltpu.VMEM((2 * _KT * 128, _FEAT), jnp.float32),
            pltpu.SemaphoreType.DMA,
            pltpu.SemaphoreType.DMA,
        ],
        compiler_params=pltpu.CompilerParams(needs_layout_passes=False,
                                             use_tc_tiling_on_sc=False),
    )
    return f(src_flat)


def _sc_encode_body(coords_hbm, ht_hbm, out_hbm,
                    crd_v, fr_v, idx_v, rows_v, feats_v, gsem, csem, wsem):
    wid = lax.axis_index("s") * _NC + lax.axis_index("c")
    wbase = wid * _PTS_PER_W
    lane = lax.iota(jnp.int32, 16)
    pat8 = lax.shift_right_logical(lane, 3)
    lane7 = lane & 7

    def fire_coords(c):
        for d in range(3):
            pltpu.async_copy(
                coords_hbm.at[pl.ds(d * _N_PTS + wbase + c * _P, _P)],
                crd_v.at[pl.ds((c & 1) * (3 * _P) + d * _P, _P)], csem)

    def drain_coords():
        for d in range(3):
            pltpu.make_async_copy(coords_hbm.at[pl.ds(0, _P)],
                                  crd_v.at[pl.ds(d * _P, _P)], csem).wait()

    fire_coords(0)

    def chunk_body(chunk, carry):
        base = wbase + chunk * _P
        cb = (chunk & 1) * (3 * _P)
        drain_coords()
        fire_coords(jnp.minimum(chunk + 1, _NCHUNK - 1))

        def grp(g, c2):
            off = g * 16
            xg = plsc.load_gather(crd_v, [cb + off + lane])
            yg = plsc.load_gather(crd_v, [cb + _P + off + lane])
            zg = plsc.load_gather(crd_v, [cb + 2 * _P + off + lane])
            for l in range(_NUM_LEVELS):
                res = float(_RES[l])
                fx = xg * res
                xi = fx.astype(jnp.int32)
                plsc.store_scatter(fr_v, [(l * 3 + 0) * _P + off + lane],
                                   fx - xi.astype(jnp.float32))
                fy = yg * res
                yi = fy.astype(jnp.int32)
                plsc.store_scatter(fr_v, [(l * 3 + 1) * _P + off + lane],
                                   fy - yi.astype(jnp.float32))
                fz = zg * res
                zi = fz.astype(jnp.int32)
                plsc.store_scatter(fr_v, [(l * 3 + 2) * _P + off + lane],
                                   fz - zi.astype(jnp.float32))
                x1 = xi + 1
                y0 = yi * _K1
                y1 = y0 + _K1
                z0 = zi * _K2
                z1 = z0 + _K2
                lt = l * _TABLE
                c = 0
                for hx in (xi, x1):
                    for hy in (y0, y1):
                        for hz in (z0, z1):
                            h = hx ^ hy ^ hz
                            plsc.store_scatter(
                                idx_v, [(l * 8 + c) * _P + off + lane],
                                (h & _MASK) + lt)
                            c += 1
            return c2
        lax.fori_loop(0, _P // 16, grp, None)

        handles = [pltpu.async_copy(ht_hbm.at[idx_v.at[pl.ds(r * _P, _P)]],
                                    rows_v.at[pl.ds(r * _P, _P)], gsem)
                   for r in range(_R)]
        for h in handles:
            h.wait()

        @pl.when(chunk > 0)
        def _():
            pltpu.make_async_copy(out_hbm.at[pl.ds(0, _FB)], feats_v,
                                  wsem).wait()

        def pair(k, c2):
            i1 = k * 2 + pat8
            fb = i1 * _ENC + lane7
            for l in range(_NUM_LEVELS):
                wx = plsc.load_gather(fr_v, [(l * 3 + 0) * _P + i1])
                wy = plsc.load_gather(fr_v, [(l * 3 + 1) * _P + i1])
                wz = plsc.load_gather(fr_v, [(l * 3 + 2) * _P + i1])
                wx0 = 1.0 - wx
                wy0 = 1.0 - wy
                wz0 = 1.0 - wz
                c00 = wy0 * wz0
                c01 = wy0 * wz
                c10 = wy * wz0
                c11 = wy * wz
                wts = (wx0 * c00, wx0 * c01, wx0 * c10, wx0 * c11,
                       wx * c00, wx * c01, wx * c10, wx * c11)
                acc = None
                for c in range(8):
                    g = plsc.load_gather(rows_v,
                                         [(l * 8 + c) * _P + i1, lane7])
                    t = g * wts[c]
                    acc = t if acc is None else acc + t
                plsc.store_scatter(feats_v, [fb + l * 8], acc)
            return c2
        lax.fori_loop(0, _P // 2, pair, None)

        pltpu.async_copy(feats_v, out_hbm.at[pl.ds(base * _ENC, _FB)], wsem)
        return carry

    lax.fori_loop(0, _NCHUNK, chunk_body, None)
    pltpu.make_async_copy(out_hbm.at[pl.ds(0, _FB)], feats_v, wsem).wait()
    drain_coords()


def _sc_encode(coords_flat, ht):
    mesh = plsc.VectorSubcoreMesh(core_axis_name="c", subcore_axis_name="s",
                                  num_cores=_NC, num_subcores=_NS)
    f = pl.kernel(
        _sc_encode_body,
        out_type=jax.ShapeDtypeStruct((_N_PTS * _ENC,), jnp.float32),
        mesh=mesh,
        scratch_types=[
            pltpu.VMEM((2 * 3 * _P,), jnp.float32),
            pltpu.VMEM((_NUM_LEVELS * 3 * _P,), jnp.float32),
            pltpu.VMEM((_R * _P,), jnp.int32),
            pltpu.VMEM((_R * _P, _FEAT), jnp.float32),
            pltpu.VMEM((_FB,), jnp.float32),
            pltpu.SemaphoreType.DMA,
            pltpu.SemaphoreType.DMA,
            pltpu.SemaphoreType.DMA,
        ],
        compiler_params=pltpu.CompilerParams(needs_layout_passes=False,
                                             use_tc_tiling_on_sc=False),
    )
    return f(coords_flat, ht)


def _mlp(feats, view_dirs, W1, b1, W2, b2, W3, b3):
    BT = 8192
    W1f = W1[:_ENC]
    W1v = W1[_ENC:]

    def body(f_ref, vd_ref, w1f, w1v, b1r, w2, b2r, w3, b3r, out_ref):
        hp = lax.Precision.HIGHEST
        f = f_ref[...]
        vd = vd_ref[...]
        h = jnp.dot(f, w1f[...], preferred_element_type=jnp.float32,
                    precision=hp)
        h = h + jnp.dot(vd, w1v[...], preferred_element_type=jnp.float32,
                        precision=hp)
        h = jnp.maximum(h + b1r[...], 0.0)
        h = jnp.dot(h, w2[...], preferred_element_type=jnp.float32,
                    precision=hp)
        h = jnp.maximum(h + b2r[...], 0.0)
        raw = jnp.dot(h, w3[...], preferred_element_type=jnp.float32,
                      precision=hp) + b3r[...]
        ci = lax.broadcasted_iota(jnp.int32, raw.shape, 1)
        out_ref[...] = jnp.where(ci < 3, jax.nn.sigmoid(raw),
                                 jnp.maximum(raw, 0.0))

    return pl.pallas_call(
        body,
        grid=(_N_PTS // BT,),
        in_specs=[
            pl.BlockSpec((BT, _ENC), lambda i: (i, 0)),
            pl.BlockSpec((BT, 3), lambda i: (i, 0)),
            pl.BlockSpec((_ENC, _HIDDEN), lambda i: (0, 0)),
            pl.BlockSpec((3, _HIDDEN), lambda i: (0, 0)),
            pl.BlockSpec((1, _HIDDEN), lambda i: (0, 0)),
            pl.BlockSpec((_HIDDEN, _HIDDEN), lambda i: (0, 0)),
            pl.BlockSpec((1, _HIDDEN), lambda i: (0, 0)),
            pl.BlockSpec((_HIDDEN, 4), lambda i: (0, 0)),
            pl.BlockSpec((1, 4), lambda i: (0, 0)),
        ],
        out_specs=pl.BlockSpec((BT, 4), lambda i: (i, 0)),
        out_shape=jax.ShapeDtypeStruct((_N_PTS, 4), jnp.float32),
    )(feats, view_dirs, W1f, W1v, b1.reshape(1, -1), W2, b2.reshape(1, -1),
      W3, b3.reshape(1, -1))


def kernel(coords, view_dirs, hash_tables, W1, b1, W2, b2, W3, b3):
    coords_flat = coords.T.reshape(-1)
    hv = hash_tables.transpose(0, 2, 1).reshape(
        _NUM_LEVELS, _FEAT, _TABLE // 128, 128)
    hv = hv.transpose(0, 2, 1, 3).reshape(-1)
    ht = _sc_retile(hv)
    feats = _sc_encode(coords_flat, ht).reshape(_N_PTS, _ENC)
    out = _mlp(feats, view_dirs, W1, b1, W2, b2, W3, b3)
    return out[:, :3], out[:, 3:4]


# speedup vs baseline: 2.0489x; 1.0159x over previous
import numpy as np
import jax
import jax.numpy as jnp
from jax import lax
from jax.experimental import pallas as pl
from jax.experimental.pallas import tpu as pltpu
from jax.experimental.pallas import tpu_sc as plsc

_NUM_LEVELS = 12
_FEAT = 8
_TABLE = 2 ** 19
_MASK = _TABLE - 1
_BASE_RES = 16
_MAX_RES = 512
_N_PTS = 65536
_HIDDEN = 64

_growth = np.exp((np.log(_MAX_RES) - np.log(_BASE_RES)) / (_NUM_LEVELS - 1))
_RES = [int(np.floor(_BASE_RES * _growth ** l)) for l in range(_NUM_LEVELS)]
_K1 = int(np.uint32(2654435761).astype(np.int32))
_K2 = 805459861

_NC, _NS = 2, 16
_NW = _NC * _NS
_P = 128
_PTS_PER_W = _N_PTS // _NW
_NCHUNK = _PTS_PER_W // _P
_R = _NUM_LEVELS * 8
_ENC = _NUM_LEVELS * _FEAT
_FB = _P * _ENC

_TILES = _NUM_LEVELS * (_TABLE // 128)
_TPW = _TILES // _NW
_KT = 24
_NIT = _TPW // _KT


def _sc_retile_body(src_hbm, out_hbm, bin_v, bout_v, csem, wsem):
    wid = lax.axis_index("s") * _NC + lax.axis_index("c")
    lane = lax.iota(jnp.int32, 16)
    pat8 = lax.shift_right_logical(lane, 3)
    lane7 = lane & 7
    patf = lane7 * 128 + pat8
    t0w = wid * _TPW

    def fire(it):
        pltpu.async_copy(
            src_hbm.at[pl.ds((t0w + it * _KT) * 1024, _KT * 1024)],
            bin_v.at[pl.ds((it & 1) * (_KT * 1024), _KT * 1024)], csem)

    fire(0)

    def it_body(it, carry):
        pltpu.make_async_copy(src_hbm.at[pl.ds(0, _KT * 1024)],
                              bin_v.at[pl.ds(0, _KT * 1024)], csem).wait()
        fire(jnp.minimum(it + 1, _NIT - 1))
        bb = (it & 1) * (_KT * 1024)
        ob = (it & 1) * (_KT * 128)

        @pl.when(it > 1)
        def _():
            pltpu.make_async_copy(
                out_hbm.at[pl.ds(0, _KT * 128)],
                bout_v.at[pl.ds(0, _KT * 128), :], wsem).wait()

        def tile_body(k, c2):
            kb = bb + k * 1024
            orow = ob + k * 128
            for c0 in range(0, 128, 2):
                v = plsc.load_gather(bin_v, [kb + c0 + patf])
                plsc.store_scatter(bout_v, [orow + c0 + pat8, lane7], v)
            return c2
        lax.fori_loop(0, _KT, tile_body, None)

        pltpu.async_copy(
            bout_v.at[pl.ds(ob, _KT * 128), :],
            out_hbm.at[pl.ds((t0w + it * _KT) * 128, _KT * 128)], wsem)
        return carry

    lax.fori_loop(0, _NIT, it_body, None)
    for _ in range(2):
        pltpu.make_async_copy(out_hbm.at[pl.ds(0, _KT * 128)],
                              bout_v.at[pl.ds(0, _KT * 128), :], wsem).wait()
    pltpu.make_async_copy(src_hbm.at[pl.ds(0, _KT * 1024)],
                          bin_v.at[pl.ds(0, _KT * 1024)], csem).wait()


def _sc_retile(src_flat):
    mesh = plsc.VectorSubcoreMesh(core_axis_name="c", subcore_axis_name="s",
                                  num_cores=_NC, num_subcores=_NS)
    f = pl.kernel(
        _sc_retile_body,
        out_type=jax.ShapeDtypeStruct((_NUM_LEVELS * _TABLE, _FEAT),
                                      jnp.float32),
        mesh=mesh,
        scratch_types=[
            pltpu.VMEM((2 * _KT * 1024,), jnp.float32),
            pltpu.VMEM((2 * _KT * 128, _FEAT), jnp.float32),
            pltpu.SemaphoreType.DMA,
            pltpu.SemaphoreType.DMA,
        ],
        compiler_params=pltpu.CompilerParams(needs_layout_passes=False,
                                             use_tc_tiling_on_sc=False),
    )
    return f(src_flat)


def _sc_encode_body(coords_hbm, ht_hbm, out_hbm,
                    crd_v, fr_v, idx_v, rows_v, feats_v, gsem, csem, wsem):
    wid = lax.axis_index("s") * _NC + lax.axis_index("c")
    wbase = wid * _PTS_PER_W
    lane = lax.iota(jnp.int32, 16)
    pat8 = lax.shift_right_logical(lane, 3)
    lane7 = lane & 7

    def fire_coords(c):
        for d in range(3):
            pltpu.async_copy(
                coords_hbm.at[pl.ds(d * _N_PTS + wbase + c * _P, _P)],
                crd_v.at[pl.ds((c & 1) * (3 * _P) + d * _P, _P)], csem)

    def drain_coords():
        for d in range(3):
            pltpu.make_async_copy(coords_hbm.at[pl.ds(0, _P)],
                                  crd_v.at[pl.ds(d * _P, _P)], csem).wait()

    fire_coords(0)

    def chunk_body(chunk, carry):
        base = wbase + chunk * _P
        cb = (chunk & 1) * (3 * _P)
        drain_coords()
        fire_coords(jnp.minimum(chunk + 1, _NCHUNK - 1))

        def grp(g, c2):
            off = g * 16
            xg = plsc.load_gather(crd_v, [cb + off + lane])
            yg = plsc.load_gather(crd_v, [cb + _P + off + lane])
            zg = plsc.load_gather(crd_v, [cb + 2 * _P + off + lane])
            for l in range(_NUM_LEVELS):
                res = float(_RES[l])
                fx = xg * res
                xi = fx.astype(jnp.int32)
                plsc.store_scatter(fr_v, [(l * 3 + 0) * _P + off + lane],
                                   fx - xi.astype(jnp.float32))
                fy = yg * res
                yi = fy.astype(jnp.int32)
                plsc.store_scatter(fr_v, [(l * 3 + 1) * _P + off + lane],
                                   fy - yi.astype(jnp.float32))
                fz = zg * res
                zi = fz.astype(jnp.int32)
                plsc.store_scatter(fr_v, [(l * 3 + 2) * _P + off + lane],
                                   fz - zi.astype(jnp.float32))
                x1 = xi + 1
                y0 = yi * _K1
                y1 = y0 + _K1
                z0 = zi * _K2
                z1 = z0 + _K2
                lt = l * _TABLE
                c = 0
                for hx in (xi, x1):
                    for hy in (y0, y1):
                        for hz in (z0, z1):
                            h = hx ^ hy ^ hz
                            plsc.store_scatter(
                                idx_v, [(l * 8 + c) * _P + off + lane],
                                (h & _MASK) + lt)
                            c += 1
            return c2
        lax.fori_loop(0, _P // 16, grp, None)

        handles = [pltpu.async_copy(ht_hbm.at[idx_v.at[pl.ds(r * _P, _P)]],
                                    rows_v.at[pl.ds(r * _P, _P)], gsem)
                   for r in range(_R)]
        for h in handles:
            h.wait()

        @pl.when(chunk > 0)
        def _():
            pltpu.make_async_copy(out_hbm.at[pl.ds(0, _FB)], feats_v,
                                  wsem).wait()

        def pair(k, c2):
            i1 = k * 2 + pat8
            fb = i1 * _ENC + lane7
            for l in range(_NUM_LEVELS):
                wx = plsc.load_gather(fr_v, [(l * 3 + 0) * _P + i1])
                wy = plsc.load_gather(fr_v, [(l * 3 + 1) * _P + i1])
                wz = plsc.load_gather(fr_v, [(l * 3 + 2) * _P + i1])
                wx0 = 1.0 - wx
                wy0 = 1.0 - wy
                wz0 = 1.0 - wz
                c00 = wy0 * wz0
                c01 = wy0 * wz
                c10 = wy * wz0
                c11 = wy * wz
                wts = (wx0 * c00, wx0 * c01, wx0 * c10, wx0 * c11,
                       wx * c00, wx * c01, wx * c10, wx * c11)
                acc = None
                for c in range(8):
                    g = plsc.load_gather(rows_v,
                                         [(l * 8 + c) * _P + i1, lane7])
                    t = g * wts[c]
                    acc = t if acc is None else acc + t
                plsc.store_scatter(feats_v, [fb + l * 8], acc)
            return c2
        lax.fori_loop(0, _P // 2, pair, None)

        pltpu.async_copy(feats_v, out_hbm.at[pl.ds(base * _ENC, _FB)], wsem)
        return carry

    lax.fori_loop(0, _NCHUNK, chunk_body, None)
    pltpu.make_async_copy(out_hbm.at[pl.ds(0, _FB)], feats_v, wsem).wait()
    drain_coords()


def _sc_encode(coords_flat, ht):
    mesh = plsc.VectorSubcoreMesh(core_axis_name="c", subcore_axis_name="s",
                                  num_cores=_NC, num_subcores=_NS)
    f = pl.kernel(
        _sc_encode_body,
        out_type=jax.ShapeDtypeStruct((_N_PTS * _ENC,), jnp.float32),
        mesh=mesh,
        scratch_types=[
            pltpu.VMEM((2 * 3 * _P,), jnp.float32),
            pltpu.VMEM((_NUM_LEVELS * 3 * _P,), jnp.float32),
            pltpu.VMEM((_R * _P,), jnp.int32),
            pltpu.VMEM((_R * _P, _FEAT), jnp.float32),
            pltpu.VMEM((_FB,), jnp.float32),
            pltpu.SemaphoreType.DMA,
            pltpu.SemaphoreType.DMA,
            pltpu.SemaphoreType.DMA,
        ],
        compiler_params=pltpu.CompilerParams(needs_layout_passes=False,
                                             use_tc_tiling_on_sc=False),
    )
    return f(coords_flat, ht)


def _mlp(feats, view_dirs, W1, b1, W2, b2, W3, b3):
    BT = 8192
    W1f = W1[:_ENC]
    W1v = W1[_ENC:]

    def body(f_ref, vd_ref, w1f, w1v, b1r, w2, b2r, w3, b3r,
             rgb_ref, den_ref):
        hp = lax.Precision.HIGHEST
        f = f_ref[...]
        vd = vd_ref[...]
        h = jnp.dot(f, w1f[...], preferred_element_type=jnp.float32,
                    precision=hp)
        h = h + jnp.dot(vd, w1v[...], preferred_element_type=jnp.float32,
                        precision=hp)
        h = jnp.maximum(h + b1r[...], 0.0)
        h = jnp.dot(h, w2[...], preferred_element_type=jnp.float32,
                    precision=hp)
        h = jnp.maximum(h + b2r[...], 0.0)
        raw = jnp.dot(h, w3[...], preferred_element_type=jnp.float32,
                      precision=hp) + b3r[...]
        rgb_ref[...] = jax.nn.sigmoid(raw[:, :3])
        den_ref[...] = jnp.maximum(raw[:, 3:4], 0.0)

    return pl.pallas_call(
        body,
        grid=(_N_PTS // BT,),
        in_specs=[
            pl.BlockSpec((BT, _ENC), lambda i: (i, 0)),
            pl.BlockSpec((BT, 3), lambda i: (i, 0)),
            pl.BlockSpec((_ENC, _HIDDEN), lambda i: (0, 0)),
            pl.BlockSpec((3, _HIDDEN), lambda i: (0, 0)),
            pl.BlockSpec((1, _HIDDEN), lambda i: (0, 0)),
            pl.BlockSpec((_HIDDEN, _HIDDEN), lambda i: (0, 0)),
            pl.BlockSpec((1, _HIDDEN), lambda i: (0, 0)),
            pl.BlockSpec((_HIDDEN, 4), lambda i: (0, 0)),
            pl.BlockSpec((1, 4), lambda i: (0, 0)),
        ],
        out_specs=[pl.BlockSpec((BT, 3), lambda i: (i, 0)),
                   pl.BlockSpec((BT, 1), lambda i: (i, 0))],
        out_shape=[jax.ShapeDtypeStruct((_N_PTS, 3), jnp.float32),
                   jax.ShapeDtypeStruct((_N_PTS, 1), jnp.float32)],
    )(feats, view_dirs, W1f, W1v, b1.reshape(1, -1), W2, b2.reshape(1, -1),
      W3, b3.reshape(1, -1))


def kernel(coords, view_dirs, hash_tables, W1, b1, W2, b2, W3, b3):
    coords_flat = coords.T.reshape(-1)
    hv = hash_tables.transpose(0, 2, 1).reshape(
        _NUM_LEVELS, _FEAT, _TABLE // 128, 128)
    hv = hv.transpose(0, 2, 1, 3).reshape(-1)
    ht = _sc_retile(hv)
    feats = _sc_encode(coords_flat, ht).reshape(_N_PTS, _ENC)
    rgb, den = _mlp(feats, view_dirs, W1, b1, W2, b2, W3, b3)
    return rgb, den


# speedup vs baseline: 2.2605x; 1.1033x over previous
import numpy as np
import jax
import jax.numpy as jnp
from jax import lax
from jax.experimental import pallas as pl
from jax.experimental.pallas import tpu as pltpu
from jax.experimental.pallas import tpu_sc as plsc

_NUM_LEVELS = 12
_FEAT = 8
_TABLE = 2 ** 19
_MASK = _TABLE - 1
_BASE_RES = 16
_MAX_RES = 512
_N_PTS = 65536
_HIDDEN = 64

_growth = np.exp((np.log(_MAX_RES) - np.log(_BASE_RES)) / (_NUM_LEVELS - 1))
_RES = [int(np.floor(_BASE_RES * _growth ** l)) for l in range(_NUM_LEVELS)]
_K1 = int(np.uint32(2654435761).astype(np.int32))
_K2 = 805459861

_NC, _NS = 2, 16
_NW = _NC * _NS
_P = 128
_PTS_PER_W = _N_PTS // _NW
_NCHUNK = _PTS_PER_W // _P
_R = _NUM_LEVELS * 8
_ENC = _NUM_LEVELS * _FEAT
_FB = _P * _ENC

_TILES = _NUM_LEVELS * (_TABLE // 128)
_TPW = _TILES // _NW
_KT = 24
_NIT = _TPW // _KT


def _sc_retile_body(src_hbm, out_hbm, bin_v, bout_v, csem, wsem):
    wid = lax.axis_index("s") * _NC + lax.axis_index("c")
    lane = lax.iota(jnp.int32, 16)
    pat8 = lax.shift_right_logical(lane, 3)
    lane7 = lane & 7
    patf = lane7 * 128 + pat8
    t0w = wid * _TPW

    def fire(it):
        pltpu.async_copy(
            src_hbm.at[pl.ds((t0w + it * _KT) * 1024, _KT * 1024)],
            bin_v.at[pl.ds((it & 1) * (_KT * 1024), _KT * 1024)], csem)

    fire(0)

    def it_body(it, carry):
        pltpu.make_async_copy(src_hbm.at[pl.ds(0, _KT * 1024)],
                              bin_v.at[pl.ds(0, _KT * 1024)], csem).wait()
        fire(jnp.minimum(it + 1, _NIT - 1))
        bb = (it & 1) * (_KT * 1024)
        ob = (it & 1) * (_KT * 128)

        @pl.when(it > 1)
        def _():
            pltpu.make_async_copy(
                out_hbm.at[pl.ds(0, _KT * 128)],
                bout_v.at[pl.ds(0, _KT * 128), :], wsem).wait()

        def tile_body(k, c2):
            kb = bb + k * 1024
            orow = ob + k * 128
            for c0 in range(0, 128, 2):
                v = plsc.load_gather(bin_v, [kb + c0 + patf])
                plsc.store_scatter(bout_v, [orow + c0 + pat8, lane7], v)
            return c2
        lax.fori_loop(0, _KT, tile_body, None)

        pltpu.async_copy(
            bout_v.at[pl.ds(ob, _KT * 128), :],
            out_hbm.at[pl.ds((t0w + it * _KT) * 128, _KT * 128)], wsem)
        return carry

    lax.fori_loop(0, _NIT, it_body, None)
    for _ in range(2):
        pltpu.make_async_copy(out_hbm.at[pl.ds(0, _KT * 128)],
                              bout_v.at[pl.ds(0, _KT * 128), :], wsem).wait()
    pltpu.make_async_copy(src_hbm.at[pl.ds(0, _KT * 1024)],
                          bin_v.at[pl.ds(0, _KT * 1024)], csem).wait()


def _sc_retile(src_flat):
    mesh = plsc.VectorSubcoreMesh(core_axis_name="c", subcore_axis_name="s",
                                  num_cores=_NC, num_subcores=_NS)
    f = pl.kernel(
        _sc_retile_body,
        out_type=jax.ShapeDtypeStruct((_NUM_LEVELS * _TABLE, _FEAT),
                                      jnp.float32),
        mesh=mesh,
        scratch_types=[
            pltpu.VMEM((2 * _KT * 1024,), jnp.float32),
            pltpu.VMEM((2 * _KT * 128, _FEAT), jnp.float32),
            pltpu.SemaphoreType.DMA,
            pltpu.SemaphoreType.DMA,
        ],
        compiler_params=pltpu.CompilerParams(needs_layout_passes=False,
                                             use_tc_tiling_on_sc=False),
    )
    return f(src_flat)


def _sc_encode_body(coords_hbm, ht_hbm, out_hbm,
                    crd_v, fr_v, idx_v, rows_v, feats_v, gsem, csem, wsem):
    wid = lax.axis_index("s") * _NC + lax.axis_index("c")
    wbase = wid * _PTS_PER_W
    lane = lax.iota(jnp.int32, 16)
    pat8 = lax.shift_right_logical(lane, 3)
    lane7 = lane & 7

    def fire_coords(c):
        for d in range(3):
            pltpu.async_copy(
                coords_hbm.at[pl.ds(d * _N_PTS + wbase + c * _P, _P)],
                crd_v.at[pl.ds((c & 1) * (3 * _P) + d * _P, _P)], csem)

    def drain_coords():
        for d in range(3):
            pltpu.make_async_copy(coords_hbm.at[pl.ds(0, _P)],
                                  crd_v.at[pl.ds(d * _P, _P)], csem).wait()

    fire_coords(0)

    def chunk_body(chunk, carry):
        base = wbase + chunk * _P
        cb = (chunk & 1) * (3 * _P)
        drain_coords()
        fire_coords(jnp.minimum(chunk + 1, _NCHUNK - 1))

        def grp(g, c2):
            off = g * 16
            xg = plsc.load_gather(crd_v, [cb + off + lane])
            yg = plsc.load_gather(crd_v, [cb + _P + off + lane])
            zg = plsc.load_gather(crd_v, [cb + 2 * _P + off + lane])
            for l in range(_NUM_LEVELS):
                res = float(_RES[l])
                fx = xg * res
                xi = fx.astype(jnp.int32)
                plsc.store_scatter(fr_v, [(l * 3 + 0) * _P + off + lane],
                                   fx - xi.astype(jnp.float32))
                fy = yg * res
                yi = fy.astype(jnp.int32)
                plsc.store_scatter(fr_v, [(l * 3 + 1) * _P + off + lane],
                                   fy - yi.astype(jnp.float32))
                fz = zg * res
                zi = fz.astype(jnp.int32)
                plsc.store_scatter(fr_v, [(l * 3 + 2) * _P + off + lane],
                                   fz - zi.astype(jnp.float32))
                x1 = xi + 1
                y0 = yi * _K1
                y1 = y0 + _K1
                z0 = zi * _K2
                z1 = z0 + _K2
                lt = l * _TABLE
                c = 0
                for hx in (xi, x1):
                    for hy in (y0, y1):
                        for hz in (z0, z1):
                            h = hx ^ hy ^ hz
                            plsc.store_scatter(
                                idx_v, [(l * 8 + c) * _P + off + lane],
                                (h & _MASK) + lt)
                            c += 1
            return c2
        lax.fori_loop(0, _P // 16, grp, None)

        handles = [pltpu.async_copy(ht_hbm.at[idx_v.at[pl.ds(r * _P, _P)]],
                                    rows_v.at[pl.ds(r * _P, _P)], gsem)
                   for r in range(_R)]
        for h in handles:
            h.wait()

        @pl.when(chunk > 0)
        def _():
            pltpu.make_async_copy(out_hbm.at[pl.ds(0, _FB)], feats_v,
                                  wsem).wait()

        def pair(k, c2):
            i1 = k * 2 + pat8
            fb = i1 * _ENC + lane7
            for l in range(_NUM_LEVELS):
                wx = plsc.load_gather(fr_v, [(l * 3 + 0) * _P + i1])
                wy = plsc.load_gather(fr_v, [(l * 3 + 1) * _P + i1])
                wz = plsc.load_gather(fr_v, [(l * 3 + 2) * _P + i1])
                wx0 = 1.0 - wx
                wy0 = 1.0 - wy
                wz0 = 1.0 - wz
                c00 = wy0 * wz0
                c01 = wy0 * wz
                c10 = wy * wz0
                c11 = wy * wz
                wts = (wx0 * c00, wx0 * c01, wx0 * c10, wx0 * c11,
                       wx * c00, wx * c01, wx * c10, wx * c11)
                acc = None
                for c in range(8):
                    g = plsc.load_gather(rows_v,
                                         [(l * 8 + c) * _P + i1, lane7])
                    t = g * wts[c]
                    acc = t if acc is None else acc + t
                plsc.store_scatter(feats_v, [fb + l * 8], acc)
            return c2
        lax.fori_loop(0, _P // 2, pair, None)

        pltpu.async_copy(feats_v, out_hbm.at[pl.ds(base * _ENC, _FB)], wsem)
        return carry

    lax.fori_loop(0, _NCHUNK, chunk_body, None)
    pltpu.make_async_copy(out_hbm.at[pl.ds(0, _FB)], feats_v, wsem).wait()
    drain_coords()


def _sc_encode(coords_flat, ht):
    mesh = plsc.VectorSubcoreMesh(core_axis_name="c", subcore_axis_name="s",
                                  num_cores=_NC, num_subcores=_NS)
    f = pl.kernel(
        _sc_encode_body,
        out_type=jax.ShapeDtypeStruct((_N_PTS * _ENC,), jnp.float32),
        mesh=mesh,
        scratch_types=[
            pltpu.VMEM((2 * 3 * _P,), jnp.float32),
            pltpu.VMEM((_NUM_LEVELS * 3 * _P,), jnp.float32),
            pltpu.VMEM((_R * _P,), jnp.int32),
            pltpu.VMEM((_R * _P, _FEAT), jnp.float32),
            pltpu.VMEM((_FB,), jnp.float32),
            pltpu.SemaphoreType.DMA,
            pltpu.SemaphoreType.DMA,
            pltpu.SemaphoreType.DMA,
        ],
        compiler_params=pltpu.CompilerParams(needs_layout_passes=False,
                                             use_tc_tiling_on_sc=False),
    )
    return f(coords_flat, ht)


def _mlp(feats, view_dirs, W1, b1, W2, b2, W3, b3):
    BT = 8192

    def body(f_ref, vd_ref, w1, b1r, w2, b2r, w3, b3r, out_ref):
        hin = jnp.concatenate([f_ref[...], vd_ref[...]], axis=1)
        h = jnp.dot(hin, w1[...], preferred_element_type=jnp.float32)
        h = jnp.maximum(h + b1r[...], 0.0)
        h = jnp.dot(h, w2[...], preferred_element_type=jnp.float32)
        h = jnp.maximum(h + b2r[...], 0.0)
        raw = jnp.dot(h, w3[...], preferred_element_type=jnp.float32)
        raw = raw + b3r[...]
        ci = lax.broadcasted_iota(jnp.int32, raw.shape, 1)
        out_ref[...] = jnp.where(ci < 3, jax.nn.sigmoid(raw),
                                 jnp.maximum(raw, 0.0))

    return pl.pallas_call(
        body,
        grid=(_N_PTS // BT,),
        in_specs=[
            pl.BlockSpec((BT, _ENC), lambda i: (i, 0)),
            pl.BlockSpec((BT, 3), lambda i: (i, 0)),
            pl.BlockSpec((_ENC + 3, _HIDDEN), lambda i: (0, 0)),
            pl.BlockSpec((1, _HIDDEN), lambda i: (0, 0)),
            pl.BlockSpec((_HIDDEN, _HIDDEN), lambda i: (0, 0)),
            pl.BlockSpec((1, _HIDDEN), lambda i: (0, 0)),
            pl.BlockSpec((_HIDDEN, 4), lambda i: (0, 0)),
            pl.BlockSpec((1, 4), lambda i: (0, 0)),
        ],
        out_specs=pl.BlockSpec((BT, 4), lambda i: (i, 0)),
        out_shape=jax.ShapeDtypeStruct((_N_PTS, 4), jnp.float32),
    )(feats, view_dirs, W1, b1.reshape(1, -1), W2, b2.reshape(1, -1),
      W3, b3.reshape(1, -1))


def kernel(coords, view_dirs, hash_tables, W1, b1, W2, b2, W3, b3):
    coords_flat = coords.T.reshape(-1)
    hv = hash_tables.transpose(0, 2, 1).reshape(
        _NUM_LEVELS, _FEAT, _TABLE // 128, 128)
    hv = hv.transpose(0, 2, 1, 3).reshape(-1)
    ht = _sc_retile(hv)
    feats = _sc_encode(coords_flat, ht).reshape(_N_PTS, _ENC)
    out = _mlp(feats, view_dirs, W1, b1, W2, b2, W3, b3)
    return out[:, :3], out[:, 3:4]
